# Initial kernel scaffold; baseline (speedup 1.0000x reference)
import jax
import jax.numpy as jnp
from jax.experimental import pallas as pl


def kernel(x, token_embedding):
    raise NotImplementedError("write your pallas kernel here")


# speedup vs baseline: 6.2889x; 6.2889x over previous
import jax
import jax.numpy as jnp
import numpy as np
from jax import lax
from jax.experimental import pallas as pl
from jax.experimental.pallas import tpu as pltpu
from jax.experimental.pallas import tpu_sc as plsc

LATENT_DIM = 128
N_BINS = 100000
SEQ_LEN = 200
BATCH = 4096

NW = 32
CPW = BATCH // NW
HALF = SEQ_LEN // 2
NB = 4
LANES = 16
COLG = LATENT_DIM // LANES


def _sinusoids_flat():
    channels = LATENT_DIM
    max_timescale = 512
    log_inc = np.log(max_timescale) / (channels // 2 - 1)
    inv = np.exp(-log_inc * np.arange(channels // 2, dtype=np.float32))
    t = np.arange(SEQ_LEN, dtype=np.float32)[:, None] * inv[None, :]
    pos = np.concatenate([np.sin(t), np.cos(t)], axis=1).astype(np.float32)
    return pos.reshape(-1)


_POS_FLAT = _sinusoids_flat()


def _body(x_ref, pos_ref, table_ref, out_ref, idx_v, pos_v, rows_v, gsem, ssem):
    wid = lax.axis_index("s") * 2 + lax.axis_index("c")
    base_row = wid * CPW

    pltpu.sync_copy(pos_ref, pos_v)

    def stage_and_gather(c, b):
        pltpu.sync_copy(x_ref.at[base_row + c], idx_v.at[b])
        pltpu.async_copy(
            table_ref.at[idx_v.at[b, 0]], rows_v.at[b, pl.ds(0, HALF)], gsem.at[b]
        )
        pltpu.async_copy(
            table_ref.at[idx_v.at[b, 1]], rows_v.at[b, pl.ds(HALF, HALF)], gsem.at[b]
        )

    def process(c, b, prefetch):
        for h in range(2):
            pltpu.make_async_copy(
                table_ref.at[idx_v.at[b, h]],
                rows_v.at[b, pl.ds(h * HALF, HALF)],
                gsem.at[b],
            ).wait()

        buf = rows_v.at[b]

        def row_body(r, carry):
            off = r * LATENT_DIM
            for cg in range(COLG):
                val = pos_v[pl.ds(off + cg * LANES, LANES)]
                plsc.addupdate(buf.at[r, pl.ds(cg * LANES, LANES)], val)
            return carry

        lax.fori_loop(0, SEQ_LEN, row_body, 0, unroll=2)

        sc = pltpu.async_copy(buf, out_ref.at[base_row + c], ssem.at[b])
        sc.wait()
        if prefetch:
            stage_and_gather(c + NB, b)

    for b in range(NB):
        stage_and_gather(b, b)

    def outer(i, carry):
        c0 = i * NB
        for b in range(NB):
            process(c0 + b, b, True)
        return carry

    lax.fori_loop(0, CPW // NB - 1, outer, 0)

    for b in range(NB):
        process(CPW - NB + b, b, False)


@jax.jit
def _run(x3, pos_flat, table):
    kern = pl.kernel(
        _body,
        out_type=jax.ShapeDtypeStruct((BATCH, SEQ_LEN, LATENT_DIM), jnp.float32),
        mesh=plsc.VectorSubcoreMesh(core_axis_name="c", subcore_axis_name="s"),
        scratch_types=[
            pltpu.VMEM((NB, 2, HALF), jnp.int32),
            pltpu.VMEM((SEQ_LEN * LATENT_DIM,), jnp.float32),
            pltpu.VMEM((NB, SEQ_LEN, LATENT_DIM), jnp.float32),
            pltpu.SemaphoreType.DMA((NB,)),
            pltpu.SemaphoreType.DMA((NB,)),
        ],
    )
    return kern(x3, pos_flat, table)


def kernel(x, token_embedding):
    x3 = x.astype(jnp.int32).reshape(BATCH, 2, HALF)
    pos_flat = jnp.asarray(_POS_FLAT)
    return _run(x3, pos_flat, token_embedding)


# speedup vs baseline: 9.1599x; 1.4565x over previous
import jax
import jax.numpy as jnp
import numpy as np
from jax import lax
from jax.experimental import pallas as pl
from jax.experimental.pallas import tpu as pltpu
from jax.experimental.pallas import tpu_sc as plsc

LATENT_DIM = 128
N_BINS = 100000
SEQ_LEN = 200
BATCH = 4096

NW = 32
CPW = BATCH // NW
HALF = SEQ_LEN // 2
NB = 4
LANES = 16
COLG = LATENT_DIM // LANES


def _sinusoids_flat():
    channels = LATENT_DIM
    max_timescale = 512
    log_inc = np.log(max_timescale) / (channels // 2 - 1)
    inv = np.exp(-log_inc * np.arange(channels // 2, dtype=np.float32))
    t = np.arange(SEQ_LEN, dtype=np.float32)[:, None] * inv[None, :]
    pos = np.concatenate([np.sin(t), np.cos(t)], axis=1).astype(np.float32)
    return pos.reshape(-1)


_POS_FLAT = _sinusoids_flat()


def _body(x_ref, pos_ref, table_ref, out_ref, idx_v, pos_v, rows_v, gsem, ssem, isem):
    wid = lax.axis_index("s") * 2 + lax.axis_index("c")
    base_row = wid * CPW

    pltpu.sync_copy(pos_ref, pos_v)

    def idx_copy(c, b):
        return pltpu.make_async_copy(
            x_ref.at[base_row + c], idx_v.at[b], isem.at[b]
        )

    def gather_copy(c, b, h):
        return pltpu.make_async_copy(
            table_ref.at[idx_v.at[b, h]],
            rows_v.at[b, pl.ds(h * HALF, HALF)],
            gsem.at[b],
        )

    def scatter_copy(c, b):
        return pltpu.make_async_copy(
            rows_v.at[b], out_ref.at[base_row + c], ssem.at[b]
        )

    def process(c, b, wait_s, do_pref, do_idx):
        gather_copy(c, b, 0).wait()
        gather_copy(c, b, 1).wait()

        buf = rows_v.at[b]

        def row_body(r, carry):
            off = r * LATENT_DIM
            for cg in range(COLG):
                val = pos_v[pl.ds(off + cg * LANES, LANES)]
                plsc.addupdate(buf.at[r, pl.ds(cg * LANES, LANES)], val)
            return carry

        lax.fori_loop(0, SEQ_LEN, row_body, 0, unroll=2)

        scatter_copy(c, b).start()

        if do_pref:
            c2 = c + 2
            b2 = (b + 2) % NB
            idx_copy(c2, b2).wait()
            if wait_s:
                scatter_copy(c2 - NB, b2).wait()
            gather_copy(c2, b2, 0).start()
            gather_copy(c2, b2, 1).start()
        if do_idx:
            c3 = c + 3
            b3 = (b + 3) % NB
            idx_copy(c3, b3).start()

    pltpu.sync_copy(x_ref.at[base_row + 0], idx_v.at[0])
    pltpu.sync_copy(x_ref.at[base_row + 1], idx_v.at[1])
    for b in range(2):
        gather_copy(b, b, 0).start()
        gather_copy(b, b, 1).start()
    idx_copy(2, 2).start()
    idx_copy(3, 3).start()

    for b in range(NB):
        process(b, b, wait_s=(b >= 2), do_pref=True, do_idx=True)

    def outer(i, carry):
        c0 = i * NB
        for b in range(NB):
            process(c0 + b, b, True, True, True)
        return carry

    lax.fori_loop(1, CPW // NB - 1, outer, 0)

    process(124, 0, True, True, True)
    process(125, 1, True, True, False)
    process(126, 2, True, False, False)
    process(127, 3, True, False, False)

    for b in range(NB):
        scatter_copy(124 + b, b).wait()


@jax.jit
def _run(x3, pos_flat, table):
    kern = pl.kernel(
        _body,
        out_type=jax.ShapeDtypeStruct((BATCH, SEQ_LEN, LATENT_DIM), jnp.float32),
        mesh=plsc.VectorSubcoreMesh(core_axis_name="c", subcore_axis_name="s"),
        scratch_types=[
            pltpu.VMEM((NB, 2, HALF), jnp.int32),
            pltpu.VMEM((SEQ_LEN * LATENT_DIM,), jnp.float32),
            pltpu.VMEM((NB, SEQ_LEN, LATENT_DIM), jnp.float32),
            pltpu.SemaphoreType.DMA((NB,)),
            pltpu.SemaphoreType.DMA((NB,)),
            pltpu.SemaphoreType.DMA((NB,)),
        ],
    )
    return kern(x3, pos_flat, table)


def kernel(x, token_embedding):
    x3 = x.astype(jnp.int32).reshape(BATCH, 2, HALF)
    pos_flat = jnp.asarray(_POS_FLAT)
    return _run(x3, pos_flat, token_embedding)


# speedup vs baseline: 9.1845x; 1.0027x over previous
import jax
import jax.numpy as jnp
import numpy as np
from jax import lax
from jax.experimental import pallas as pl
from jax.experimental.pallas import tpu as pltpu
from jax.experimental.pallas import tpu_sc as plsc

LATENT_DIM = 128
N_BINS = 100000
SEQ_LEN = 200
BATCH = 4096

NW = 32
CPW = BATCH // NW
HALF = SEQ_LEN // 2
NB = 4
LANES = 16
COLG = LATENT_DIM // LANES


def _sinusoids_flat():
    channels = LATENT_DIM
    max_timescale = 512
    log_inc = np.log(max_timescale) / (channels // 2 - 1)
    inv = np.exp(-log_inc * np.arange(channels // 2, dtype=np.float32))
    t = np.arange(SEQ_LEN, dtype=np.float32)[:, None] * inv[None, :]
    pos = np.concatenate([np.sin(t), np.cos(t)], axis=1).astype(np.float32)
    return pos.reshape(-1)


_POS_FLAT = _sinusoids_flat()


def _body(x_ref, pos_ref, table_ref, out_ref, idx_v, pos_v, rows_v, gsem, ssem, isem):
    wid = lax.axis_index("s") * 2 + lax.axis_index("c")
    base_row = wid * CPW

    pltpu.sync_copy(pos_ref, pos_v)

    def idx_copy(c, b):
        return pltpu.make_async_copy(
            x_ref.at[base_row + c], idx_v.at[b], isem.at[b]
        )

    def gather_copy(c, b, h):
        return pltpu.make_async_copy(
            table_ref.at[idx_v.at[b, h]],
            rows_v.at[b, pl.ds(h * HALF, HALF)],
            gsem.at[b],
        )

    def scatter_copy(c, b):
        return pltpu.make_async_copy(
            rows_v.at[b], out_ref.at[base_row + c], ssem.at[b]
        )

    def process(c, b, wait_s, do_pref, do_idx):
        gather_copy(c, b, 0).wait()
        gather_copy(c, b, 1).wait()

        buf = rows_v.at[b]

        def row_body(r, carry):
            off = r * LATENT_DIM
            for cg in range(COLG):
                val = pos_v[pl.ds(off + cg * LANES, LANES)]
                plsc.addupdate(buf.at[r, pl.ds(cg * LANES, LANES)], val)
            return carry

        lax.fori_loop(0, SEQ_LEN, row_body, 0, unroll=2)

        scatter_copy(c, b).start()

        if do_pref:
            c2 = c + 2
            b2 = (b + 2) % NB
            idx_copy(c2, b2).wait()
            if wait_s:
                scatter_copy(c2 - NB, b2).wait()
            gather_copy(c2, b2, 0).start()
            gather_copy(c2, b2, 1).start()
        if do_idx:
            c3 = c + 3
            b3 = (b + 3) % NB
            idx_copy(c3, b3).start()

    pltpu.sync_copy(x_ref.at[base_row + 0], idx_v.at[0])
    pltpu.sync_copy(x_ref.at[base_row + 1], idx_v.at[1])
    for b in range(2):
        gather_copy(b, b, 0).start()
        gather_copy(b, b, 1).start()
    idx_copy(2, 2).start()
    idx_copy(3, 3).start()

    for b in range(NB):
        process(b, b, wait_s=(b >= 2), do_pref=True, do_idx=True)

    def outer(i, carry):
        c0 = i * NB
        for b in range(NB):
            process(c0 + b, b, True, True, True)
        return carry

    lax.fori_loop(1, CPW // NB - 1, outer, 0)

    process(124, 0, True, True, True)
    process(125, 1, True, True, False)
    process(126, 2, True, False, False)
    process(127, 3, True, False, False)

    for b in range(NB):
        scatter_copy(124 + b, b).wait()


@jax.jit
def _run(x3, table):
    pos_flat = jnp.asarray(_POS_FLAT)
    kern = pl.kernel(
        _body,
        out_type=jax.ShapeDtypeStruct((BATCH, SEQ_LEN, LATENT_DIM), jnp.float32),
        mesh=plsc.VectorSubcoreMesh(core_axis_name="c", subcore_axis_name="s"),
        scratch_types=[
            pltpu.VMEM((NB, 2, HALF), jnp.int32),
            pltpu.VMEM((SEQ_LEN * LATENT_DIM,), jnp.float32),
            pltpu.VMEM((NB, SEQ_LEN, LATENT_DIM), jnp.float32),
            pltpu.SemaphoreType.DMA((NB,)),
            pltpu.SemaphoreType.DMA((NB,)),
            pltpu.SemaphoreType.DMA((NB,)),
        ],
    )
    return kern(x3, pos_flat, table)


def kernel(x, token_embedding):
    x3 = x.astype(jnp.int32).reshape(BATCH, 2, HALF)
    return _run(x3, token_embedding)
